# baseline (speedup 1.0000x reference)
import functools

import jax
import jax.numpy as jnp
from jax import lax
from jax.experimental import pallas as pl
from jax.experimental.pallas import tpu as pltpu
from jax.experimental.pallas import tpu_sc as plsc


_NC = 2
_NS = 16
_NW = _NC * _NS
_LANES = 16


def _make_sc_pool(seq, batch, vocab, emb):
    cols = batch // _NW
    mesh = plsc.VectorSubcoreMesh(core_axis_name="c", subcore_axis_name="s")

    @functools.partial(
        pl.kernel,
        mesh=mesh,
        out_type=jax.ShapeDtypeStruct((batch, emb), jnp.float32),
        compiler_params=pltpu.CompilerParams(use_tc_tiling_on_sc=False),
        scratch_types=[
            pltpu.VMEM((seq, cols), jnp.int32),
            pltpu.VMEM((cols, emb), jnp.float32),
            pltpu.VMEM((cols,), jnp.int32),
            pltpu.VMEM_SHARED((_NS * cols, emb), jnp.float32),
        ],
    )
    def sc_pool(x_hbm, emb_hbm, out_hbm, idx_v, buf_v, dst_v, acc_sh):
        cid = lax.axis_index("c")
        sid = lax.axis_index("s")
        wid = sid * _NC + cid
        base = wid * cols

        pltpu.sync_copy(x_hbm.at[:, pl.ds(base, cols)], idx_v)

        for i in range(cols // _LANES):
            dst_v[pl.ds(i * _LANES, _LANES)] = (
                lax.iota(jnp.int32, _LANES) + (sid * cols + i * _LANES)
            )

        pltpu.sync_copy(emb_hbm.at[idx_v.at[0]], buf_v)
        pltpu.sync_copy(buf_v, acc_sh.at[dst_v])

        def step(s, carry):
            pltpu.sync_copy(emb_hbm.at[idx_v.at[s]], buf_v)
            pltpu.sync_copy(buf_v, acc_sh.at[dst_v], add=True)
            return carry

        lax.fori_loop(1, seq, step, 0)

        pltpu.sync_copy(acc_sh.at[pl.ds(sid * cols, cols)],
                        out_hbm.at[pl.ds(base, cols)])

    return sc_pool


def _tc_head(sums, fc_w, fc_b2d, seq, blk):
    batch, emb = sums.shape
    out_dim = fc_w.shape[0]
    inv = 1.0 / seq

    def body(s_ref, w_ref, b_ref, o_ref):
        s = s_ref[...]
        w = w_ref[...]
        logits = lax.dot_general(
            s, w, (((1,), (1,)), ((), ())),
            preferred_element_type=jnp.float32,
        ) * inv + b_ref[...]
        m = jnp.max(logits, axis=-1, keepdims=True)
        e = jnp.exp(logits - m)
        lse = jnp.log(jnp.sum(e, axis=-1, keepdims=True)) + m
        o_ref[...] = logits - lse

    return pl.pallas_call(
        body,
        grid=(batch // blk,),
        in_specs=[
            pl.BlockSpec((blk, emb), lambda i: (i, 0)),
            pl.BlockSpec((out_dim, emb), lambda i: (0, 0)),
            pl.BlockSpec((1, out_dim), lambda i: (0, 0)),
        ],
        out_specs=pl.BlockSpec((blk, out_dim), lambda i: (i, 0)),
        out_shape=jax.ShapeDtypeStruct((batch, out_dim), jnp.float32),
    )(sums, fc_w, fc_b2d)


def kernel(x, embedding, fc_w, fc_b):
    seq, batch = x.shape
    vocab, emb = embedding.shape
    sums = _make_sc_pool(seq, batch, vocab, emb)(x, embedding)
    return _tc_head(sums, fc_w, fc_b.reshape(1, -1), seq, blk=512)


# speedup vs baseline: 1.2244x; 1.2244x over previous
import functools

import jax
import jax.numpy as jnp
from jax import lax
from jax.experimental import pallas as pl
from jax.experimental.pallas import tpu as pltpu
from jax.experimental.pallas import tpu_sc as plsc


_NC = 2
_NS = 16
_NW = _NC * _NS
_LANES = 16


def _make_sc_pool(seq, batch, vocab, emb):
    cols = batch // _NW
    mesh = plsc.VectorSubcoreMesh(core_axis_name="c", subcore_axis_name="s")

    nbuf = 8
    assert seq % nbuf == 0

    @functools.partial(
        pl.kernel,
        mesh=mesh,
        out_type=jax.ShapeDtypeStruct((batch, emb), jnp.float32),
        compiler_params=pltpu.CompilerParams(use_tc_tiling_on_sc=False),
        scratch_types=[
            pltpu.VMEM((seq, cols), jnp.int32),
            pltpu.VMEM((nbuf, cols, emb), jnp.float32),
            pltpu.VMEM((cols,), jnp.int32),
            pltpu.VMEM_SHARED((_NS * cols, emb), jnp.float32),
            pltpu.SemaphoreType.DMA((nbuf,)),
            pltpu.SemaphoreType.DMA((nbuf,)),
        ],
    )
    def sc_pool(x_hbm, emb_hbm, out_hbm, idx_v, buf_v, dst_v, acc_sh,
                gsem, ssem):
        cid = lax.axis_index("c")
        sid = lax.axis_index("s")
        wid = sid * _NC + cid
        base = wid * cols

        pltpu.sync_copy(x_hbm.at[:, pl.ds(base, cols)], idx_v)

        for i in range(cols // _LANES):
            dst_v[pl.ds(i * _LANES, _LANES)] = (
                lax.iota(jnp.int32, _LANES) + (sid * cols + i * _LANES)
            )

        zeros = jnp.zeros((_LANES,), jnp.float32)

        def zrow(i, carry):
            for d in range(emb // _LANES):
                buf_v[0, i, pl.ds(d * _LANES, _LANES)] = zeros
            return carry

        lax.fori_loop(0, cols, zrow, 0)
        pltpu.sync_copy(buf_v.at[0], acc_sh.at[pl.ds(sid * cols, cols)])

        for b in range(nbuf):
            pltpu.async_copy(emb_hbm.at[idx_v.at[b]], buf_v.at[b], gsem.at[b])

        def group(g, carry):
            for b in range(nbuf):
                s = g * nbuf + b
                pltpu.make_async_copy(
                    emb_hbm.at[idx_v.at[s]], buf_v.at[b], gsem.at[b]).wait()
                pltpu.async_copy(buf_v.at[b], acc_sh.at[dst_v], ssem.at[b],
                                 add=True)
                pltpu.make_async_copy(
                    buf_v.at[b], acc_sh.at[dst_v], ssem.at[b]).wait()

                @pl.when(s + nbuf < seq)
                def _():
                    pltpu.async_copy(emb_hbm.at[idx_v.at[s + nbuf]],
                                     buf_v.at[b], gsem.at[b])
            return carry

        lax.fori_loop(0, seq // nbuf, group, 0)

        pltpu.sync_copy(acc_sh.at[pl.ds(sid * cols, cols)],
                        out_hbm.at[pl.ds(base, cols)])

    return sc_pool


def _tc_head(sums, fc_w, fc_b2d, seq, blk):
    batch, emb = sums.shape
    out_dim = fc_w.shape[0]
    inv = 1.0 / seq

    def body(s_ref, w_ref, b_ref, o_ref):
        s = s_ref[...]
        w = w_ref[...]
        logits = lax.dot_general(
            s, w, (((1,), (1,)), ((), ())),
            preferred_element_type=jnp.float32,
        ) * inv + b_ref[...]
        m = jnp.max(logits, axis=-1, keepdims=True)
        e = jnp.exp(logits - m)
        lse = jnp.log(jnp.sum(e, axis=-1, keepdims=True)) + m
        o_ref[...] = logits - lse

    return pl.pallas_call(
        body,
        grid=(batch // blk,),
        in_specs=[
            pl.BlockSpec((blk, emb), lambda i: (i, 0)),
            pl.BlockSpec((out_dim, emb), lambda i: (0, 0)),
            pl.BlockSpec((1, out_dim), lambda i: (0, 0)),
        ],
        out_specs=pl.BlockSpec((blk, out_dim), lambda i: (i, 0)),
        out_shape=jax.ShapeDtypeStruct((batch, out_dim), jnp.float32),
    )(sums, fc_w, fc_b2d)


def kernel(x, embedding, fc_w, fc_b):
    seq, batch = x.shape
    vocab, emb = embedding.shape
    sums = _make_sc_pool(seq, batch, vocab, emb)(x, embedding)
    return _tc_head(sums, fc_w, fc_b.reshape(1, -1), seq, blk=512)
